# baseline (speedup 1.0000x reference)
import functools

import numpy as np
import jax
import jax.numpy as jnp
from jax import lax
from jax.experimental import pallas as pl
from jax.experimental.pallas import tpu as pltpu
from jax.experimental.pallas import tpu_sc as plsc

_NUM_LEVELS = 16
_MAX_PARAMS = 2 ** 19
_MASK19 = _MAX_PARAMS - 1
_BASE = np.array([16.0, 16.0, 16.0, 16.0])
_DESIRED = np.array([1024.0, 1024.0, 1024.0, 128.0])
_PLS = np.exp2(np.log2(_DESIRED / _BASE) / (_NUM_LEVELS - 1))


def _wrap_i32(x):
    return int(np.int32(np.uint32(x & 0xFFFFFFFF)))


_P1 = _wrap_i32(2654435761)
_P2 = _wrap_i32(805459861)
_P3 = _wrap_i32(3674653429)


def _level_metas(dims):
    metas = []
    offset = 0
    for i in range(_NUM_LEVELS):
        res = np.ceil(_BASE[:dims] * _PLS[:dims] ** i).astype(np.int64) + 1
        params = int(min(_MAX_PARAMS, int(np.prod(res))))
        params = int(np.ceil(params / 8) * 8)
        dense = int(np.prod(res)) <= params
        metas.append((tuple(int(r) for r in res), params, offset, dense))
        offset += params
    return metas, offset


_SME, _S_TOTAL = _level_metas(3)
_TME, _T_TOTAL = _level_metas(4)

_B = 262144
_NW = 32
_C = 64
_PPW = _B // _NW
_NCHUNK = _PPW // _C
_NJ = _C // 16
_NL = _NUM_LEVELS

_mesh = plsc.VectorSubcoreMesh(core_axis_name="c", subcore_axis_name="s")


@functools.partial(
    pl.kernel,
    mesh=_mesh,
    out_type=jax.ShapeDtypeStruct((32, _B), jnp.float32),
    scratch_types=[
        pltpu.VMEM((4 * _C,), jnp.float32),
        pltpu.VMEM((_C,), jnp.float32),
        pltpu.VMEM((8 * _C,), jnp.int32),
        pltpu.VMEM((8 * _C,), jnp.float32),
        pltpu.VMEM((8 * _C,), jnp.float32),
        pltpu.VMEM((_NL * 8 * _C,), jnp.int32),
        pltpu.VMEM((_NL * 8 * _C,), jnp.uint32),
        pltpu.VMEM((_NL * 16 * _C,), jnp.int32),
        pltpu.VMEM((_NL * 16 * _C,), jnp.uint32),
        pltpu.VMEM((_NL * 8 * _C,), jnp.float32),
        pltpu.VMEM((_NL * 2 * _C,), jnp.float32),
        pltpu.VMEM((32, 2 * _C), jnp.float32),
        pltpu.SemaphoreType.DMA,
        pltpu.SemaphoreType.DMA,
        pltpu.SemaphoreType.DMA,
    ],
)
def _encode(x0, x1, x2, x3, spk, tpk, memb, out,
            xin, mbuf, midx, mw, mrows, sidx, svp, tidx, tvp,
            w3, wt, obuf, sem_s, sem_t, sem_m):
    wid = lax.axis_index("s") * 2 + lax.axis_index("c")

    def chunk(g, carry):
        base = wid * _PPW + g * _C
        pltpu.sync_copy(x0.at[pl.ds(base, _C)], xin.at[pl.ds(0, _C)])
        pltpu.sync_copy(x1.at[pl.ds(base, _C)], xin.at[pl.ds(_C, _C)])
        pltpu.sync_copy(x2.at[pl.ds(base, _C)], xin.at[pl.ds(2 * _C, _C)])
        pltpu.sync_copy(x3.at[pl.ds(base, _C)], xin.at[pl.ds(3 * _C, _C)])

        def idx_body(j, c2):
            j16 = j * 16
            xv = [xin[pl.ds(d * _C + j16, 16)] for d in range(4)]

            mpg, mfr = [], []
            for d in range(3):
                pos = xv[d] * 127.0
                pgi = pos.astype(jnp.int32)
                mfr.append(pos - pgi.astype(jnp.float32))
                mpg.append(pgi)
            mc1 = [jnp.minimum(mpg[d] + 1, 127) for d in range(3)]
            mu = [1.0 - f for f in mfr]
            m1 = (mpg[1] * 128, mc1[1] * 128)
            m2 = (mpg[2] * 16384, mc1[2] * 16384)
            mwxy = (mu[0] * mu[1], mfr[0] * mu[1], mu[0] * mfr[1],
                    mfr[0] * mfr[1])
            for corner in range(8):
                b0, b1, b2 = corner & 1, (corner >> 1) & 1, (corner >> 2) & 1
                iv = (mc1[0] if b0 else mpg[0]) + m1[b1] + m2[b2]
                wv = mwxy[b0 + 2 * b1] * (mfr[2] if b2 else mu[2])
                midx[pl.ds(corner * _C + j16, 16)] = iv
                mw[pl.ds(corner * _C + j16, 16)] = wv

            for l in range(_NL):
                sres, _, soff, sdense = _SME[l]
                tres, _, toff, tdense = _TME[l]
                pg, fr = [], []
                for d in range(3):
                    pos = xv[d] * float(sres[d] - 1) + 0.5
                    pgi = pos.astype(jnp.int32)
                    fr.append(pos - pgi.astype(jnp.float32))
                    pg.append(pgi)
                post = xv[3] * float(tres[3] - 1) + 0.5
                pgt = post.astype(jnp.int32)
                frt = post - pgt.astype(jnp.float32)
                cd1 = [jnp.minimum(pg[d] + 1, sres[d] - 1) for d in range(3)]
                ct1 = jnp.minimum(pgt + 1, tres[3] - 1)
                u = [1.0 - f for f in fr]
                ut = 1.0 - frt
                wxy = (u[0] * u[1], fr[0] * u[1], u[0] * fr[1], fr[0] * fr[1])
                wt[pl.ds((l * 2) * _C + j16, 16)] = ut
                wt[pl.ds((l * 2 + 1) * _C + j16, 16)] = frt

                if sdense or tdense:
                    l1 = (pg[1] * sres[0], cd1[1] * sres[0])
                    s01 = sres[0] * sres[1]
                    l2 = (pg[2] * s01, cd1[2] * s01)
                if (not sdense) or (not tdense):
                    h1 = (pg[1] * _P1, cd1[1] * _P1)
                    h2 = (pg[2] * _P2, cd1[2] * _P2)
                if tdense:
                    tstr = sres[0] * sres[1] * sres[2]
                    tm = (pgt * tstr, ct1 * tstr)
                else:
                    ht = (pgt * _P3, ct1 * _P3)

                for corner in range(8):
                    b0 = corner & 1
                    b1 = (corner >> 1) & 1
                    b2 = (corner >> 2) & 1
                    cx = cd1[0] if b0 else pg[0]
                    w3[pl.ds((l * 8 + corner) * _C + j16, 16)] = (
                        wxy[b0 + 2 * b1] * (fr[2] if b2 else u[2]))
                    if sdense or tdense:
                        lin = cx + l1[b1] + l2[b2]
                    if (not sdense) or (not tdense):
                        hsh = cx ^ h1[b1] ^ h2[b2]
                    if sdense:
                        si = lin + soff
                    else:
                        si = (hsh & _MASK19) + soff
                    sidx[pl.ds((l * 8 + corner) * _C + j16, 16)] = si
                    if tdense:
                        ti0 = lin + tm[0] + toff
                        ti1 = lin + tm[1] + toff
                    else:
                        ti0 = ((hsh ^ ht[0]) & _MASK19) + toff
                        ti1 = ((hsh ^ ht[1]) & _MASK19) + toff
                    tidx[pl.ds((l * 16 + corner) * _C + j16, 16)] = ti0
                    tidx[pl.ds((l * 16 + 8 + corner) * _C + j16, 16)] = ti1
            return c2

        lax.fori_loop(0, _NJ, idx_body, 0)

        cm = pltpu.async_copy(memb.at[midx], mrows, sem_m)
        ca = pltpu.async_copy(spk.at[sidx], svp, sem_s)
        cb = pltpu.async_copy(tpk.at[tidx], tvp, sem_t)
        cm.wait()
        ca.wait()
        cb.wait()

        def m_body(j, c2):
            j16 = j * 16
            macc = jnp.zeros((16,), jnp.float32)
            for corner in range(8):
                macc = macc + (mw[pl.ds(corner * _C + j16, 16)]
                               * mrows[pl.ds(corner * _C + j16, 16)])
            mbuf[pl.ds(j16, 16)] = 1.0 / (1.0 + jnp.exp(-macc))
            return c2

        lax.fori_loop(0, _NJ, m_body, 0)

        ocol = (g & 1) * _C

        def acc_level(l, c2):
            def acc_j(j, c3):
                j16 = j * 16
                mv = mbuf[pl.ds(j16, 16)]
                omv = 1.0 - mv
                wt0 = wt[pl.ds((l * 2) * _C + j16, 16)]
                wt1 = wt[pl.ds((l * 2 + 1) * _C + j16, 16)]
                a0 = jnp.zeros((16,), jnp.float32)
                a1 = jnp.zeros((16,), jnp.float32)
                b0 = jnp.zeros((16,), jnp.float32)
                b1 = jnp.zeros((16,), jnp.float32)
                for corner in range(8):
                    sb = (l * 8 + corner) * _C + j16
                    wv = w3[pl.ds(sb, 16)]
                    sp = svp[pl.ds(sb, 16)]
                    a0 = a0 + wv * lax.bitcast_convert_type(sp << 16, jnp.float32)
                    a1 = a1 + wv * lax.bitcast_convert_type(sp & jnp.uint32(0xFFFF0000), jnp.float32)
                    tb = (l * 16 + corner) * _C + j16
                    tp0 = tvp[pl.ds(tb, 16)]
                    tp1 = tvp[pl.ds(tb + 8 * _C, 16)]
                    r00 = lax.bitcast_convert_type(tp0 << 16, jnp.float32)
                    r01 = lax.bitcast_convert_type(tp0 & jnp.uint32(0xFFFF0000), jnp.float32)
                    r10 = lax.bitcast_convert_type(tp1 << 16, jnp.float32)
                    r11 = lax.bitcast_convert_type(tp1 & jnp.uint32(0xFFFF0000), jnp.float32)
                    b0 = b0 + wv * (wt0 * r00 + wt1 * r10)
                    b1 = b1 + wv * (wt0 * r01 + wt1 * r11)
                o0 = omv * a0 + mv * b0
                o1 = omv * a1 + mv * b1
                obuf[2 * l, pl.ds(ocol + j16, 16)] = o0
                obuf[2 * l + 1, pl.ds(ocol + j16, 16)] = o1
                return c3

            return lax.fori_loop(0, _NJ, acc_j, c2)

        lax.fori_loop(0, _NL, acc_level, 0)

        @pl.when((g & 1) == 1)
        def _flush():
            ob = pl.multiple_of(base - _C, 2 * _C)
            pltpu.sync_copy(obuf, out.at[:, pl.ds(ob, 2 * _C)])

        return carry

    lax.fori_loop(0, _NCHUNK, chunk, 0)


def _pack_pairs(table):
    c0 = jax.lax.bitcast_convert_type(
        table[:, 0].astype(jnp.bfloat16), jnp.uint16).astype(jnp.uint32)
    c1 = jax.lax.bitcast_convert_type(
        table[:, 1].astype(jnp.bfloat16), jnp.uint16).astype(jnp.uint32)
    return c0 | (c1 << 16)


def kernel(inputs, sembeddings, tembeddings, membeddings):
    x0 = inputs[:, 0]
    x1 = inputs[:, 1]
    x2 = inputs[:, 2]
    x3 = inputs[:, 3]
    spk = _pack_pairs(sembeddings)
    tpk = _pack_pairs(tembeddings)
    cm = _encode(x0, x1, x2, x3, spk, tpk, membeddings)
    return cm.T


# speedup vs baseline: 1.0087x; 1.0087x over previous
import functools

import numpy as np
import jax
import jax.numpy as jnp
from jax import lax
from jax.experimental import pallas as pl
from jax.experimental.pallas import tpu as pltpu
from jax.experimental.pallas import tpu_sc as plsc

_NUM_LEVELS = 16
_MAX_PARAMS = 2 ** 19
_MASK19 = _MAX_PARAMS - 1
_BASE = np.array([16.0, 16.0, 16.0, 16.0])
_DESIRED = np.array([1024.0, 1024.0, 1024.0, 128.0])
_PLS = np.exp2(np.log2(_DESIRED / _BASE) / (_NUM_LEVELS - 1))


def _wrap_i32(x):
    return int(np.int32(np.uint32(x & 0xFFFFFFFF)))


_P1 = _wrap_i32(2654435761)
_P2 = _wrap_i32(805459861)
_P3 = _wrap_i32(3674653429)


def _level_metas(dims):
    metas = []
    offset = 0
    for i in range(_NUM_LEVELS):
        res = np.ceil(_BASE[:dims] * _PLS[:dims] ** i).astype(np.int64) + 1
        params = int(min(_MAX_PARAMS, int(np.prod(res))))
        params = int(np.ceil(params / 8) * 8)
        dense = int(np.prod(res)) <= params
        metas.append((tuple(int(r) for r in res), params, offset, dense))
        offset += params
    return metas, offset


_SME, _S_TOTAL = _level_metas(3)
_TME, _T_TOTAL = _level_metas(4)

_B = 262144
_NW = 32
_C = 32
_PPW = _B // _NW
_NCHUNK = _PPW // _C
_NJ = _C // 16
_NL = _NUM_LEVELS

_NM = 8 * _C
_NS = _NL * 8 * _C
_NT = _NL * 16 * _C
_NWT = _NL * 2 * _C

_HI = np.uint32(0xFFFF0000)

_mesh = plsc.VectorSubcoreMesh(core_axis_name="c", subcore_axis_name="s")


@functools.partial(
    pl.kernel,
    mesh=_mesh,
    out_type=jax.ShapeDtypeStruct((32, _B), jnp.float32),
    scratch_types=[
        pltpu.VMEM((4 * _C,), jnp.float32),
        pltpu.VMEM((_C,), jnp.float32),
        pltpu.VMEM((2 * _NM,), jnp.int32),
        pltpu.VMEM((2 * _NM,), jnp.float32),
        pltpu.VMEM((2 * _NM,), jnp.float32),
        pltpu.VMEM((2 * _NS,), jnp.int32),
        pltpu.VMEM((2 * _NS,), jnp.uint32),
        pltpu.VMEM((2 * _NT,), jnp.int32),
        pltpu.VMEM((2 * _NT,), jnp.uint32),
        pltpu.VMEM((2 * _NS,), jnp.float32),
        pltpu.VMEM((2 * _NWT,), jnp.float32),
        pltpu.VMEM((32, 4 * _C), jnp.float32),
        pltpu.SemaphoreType.DMA,
        pltpu.SemaphoreType.DMA,
        pltpu.SemaphoreType.DMA,
    ],
)
def _encode(x0, x1, x2, x3, spk, tpk, memb, out,
            xin, mbuf, midx, mw, mrows, sidx, svp, tidx, tvp,
            w3, wt, obuf, sem_s, sem_t, sem_m):
    wid = lax.axis_index("s") * 2 + lax.axis_index("c")

    def idx_phase(g):
        bnk = g & 1
        bo8 = bnk * _NM
        boS = bnk * _NS
        boT = bnk * _NT
        boW = bnk * _NWT
        base = wid * _PPW + g * _C
        pltpu.sync_copy(x0.at[pl.ds(base, _C)], xin.at[pl.ds(0, _C)])
        pltpu.sync_copy(x1.at[pl.ds(base, _C)], xin.at[pl.ds(_C, _C)])
        pltpu.sync_copy(x2.at[pl.ds(base, _C)], xin.at[pl.ds(2 * _C, _C)])
        pltpu.sync_copy(x3.at[pl.ds(base, _C)], xin.at[pl.ds(3 * _C, _C)])

        def idx_body(j, c2):
            j16 = j * 16
            xv = [xin[pl.ds(d * _C + j16, 16)] for d in range(4)]

            mpg, mfr = [], []
            for d in range(3):
                pos = xv[d] * 127.0
                pgi = pos.astype(jnp.int32)
                mfr.append(pos - pgi.astype(jnp.float32))
                mpg.append(pgi)
            mc1 = [jnp.minimum(mpg[d] + 1, 127) for d in range(3)]
            mu = [1.0 - f for f in mfr]
            m1 = (mpg[1] * 128, mc1[1] * 128)
            m2 = (mpg[2] * 16384, mc1[2] * 16384)
            mwxy = (mu[0] * mu[1], mfr[0] * mu[1], mu[0] * mfr[1],
                    mfr[0] * mfr[1])
            for corner in range(8):
                b0, b1, b2 = corner & 1, (corner >> 1) & 1, (corner >> 2) & 1
                iv = (mc1[0] if b0 else mpg[0]) + m1[b1] + m2[b2]
                wv = mwxy[b0 + 2 * b1] * (mfr[2] if b2 else mu[2])
                midx[pl.ds(bo8 + corner * _C + j16, 16)] = iv
                mw[pl.ds(bo8 + corner * _C + j16, 16)] = wv

            for l in range(_NL):
                sres, _, soff, sdense = _SME[l]
                tres, _, toff, tdense = _TME[l]
                pg, fr = [], []
                for d in range(3):
                    pos = xv[d] * float(sres[d] - 1) + 0.5
                    pgi = pos.astype(jnp.int32)
                    fr.append(pos - pgi.astype(jnp.float32))
                    pg.append(pgi)
                post = xv[3] * float(tres[3] - 1) + 0.5
                pgt = post.astype(jnp.int32)
                frt = post - pgt.astype(jnp.float32)
                cd1 = [jnp.minimum(pg[d] + 1, sres[d] - 1) for d in range(3)]
                ct1 = jnp.minimum(pgt + 1, tres[3] - 1)
                u = [1.0 - f for f in fr]
                ut = 1.0 - frt
                wxy = (u[0] * u[1], fr[0] * u[1], u[0] * fr[1], fr[0] * fr[1])
                wt[pl.ds(boW + (l * 2) * _C + j16, 16)] = ut
                wt[pl.ds(boW + (l * 2 + 1) * _C + j16, 16)] = frt

                if sdense or tdense:
                    l1 = (pg[1] * sres[0], cd1[1] * sres[0])
                    s01 = sres[0] * sres[1]
                    l2 = (pg[2] * s01, cd1[2] * s01)
                if (not sdense) or (not tdense):
                    h1 = (pg[1] * _P1, cd1[1] * _P1)
                    h2 = (pg[2] * _P2, cd1[2] * _P2)
                if tdense:
                    tstr = sres[0] * sres[1] * sres[2]
                    tm = (pgt * tstr, ct1 * tstr)
                else:
                    ht = (pgt * _P3, ct1 * _P3)

                for corner in range(8):
                    b0 = corner & 1
                    b1 = (corner >> 1) & 1
                    b2 = (corner >> 2) & 1
                    cx = cd1[0] if b0 else pg[0]
                    w3[pl.ds(boS + (l * 8 + corner) * _C + j16, 16)] = (
                        wxy[b0 + 2 * b1] * (fr[2] if b2 else u[2]))
                    if sdense or tdense:
                        lin = cx + l1[b1] + l2[b2]
                    if (not sdense) or (not tdense):
                        hsh = cx ^ h1[b1] ^ h2[b2]
                    if sdense:
                        si = lin + soff
                    else:
                        si = (hsh & _MASK19) + soff
                    sidx[pl.ds(boS + (l * 8 + corner) * _C + j16, 16)] = si
                    if tdense:
                        ti0 = lin + tm[0] + toff
                        ti1 = lin + tm[1] + toff
                    else:
                        ti0 = ((hsh ^ ht[0]) & _MASK19) + toff
                        ti1 = ((hsh ^ ht[1]) & _MASK19) + toff
                    tidx[pl.ds(boT + (l * 16 + corner) * _C + j16, 16)] = ti0
                    tidx[pl.ds(boT + (l * 16 + 8 + corner) * _C + j16,
                               16)] = ti1
            return c2

        lax.fori_loop(0, _NJ, idx_body, 0)

    def _dma_args(g):
        bnk = g & 1
        return (
            (memb.at[midx.at[pl.ds(bnk * _NM, _NM)]],
             mrows.at[pl.ds(bnk * _NM, _NM)], sem_m),
            (spk.at[sidx.at[pl.ds(bnk * _NS, _NS)]],
             svp.at[pl.ds(bnk * _NS, _NS)], sem_s),
            (tpk.at[tidx.at[pl.ds(bnk * _NT, _NT)]],
             tvp.at[pl.ds(bnk * _NT, _NT)], sem_t),
        )

    def fire(g):
        for args in _dma_args(g):
            pltpu.async_copy(*args)

    def drain(g):
        for args in _dma_args(g):
            pltpu.make_async_copy(*args).wait()

    def compute_phase(g):
        bnk = g & 1
        bo8 = bnk * _NM
        boS = bnk * _NS
        boT = bnk * _NT
        boW = bnk * _NWT

        def m_body(j, c2):
            j16 = j * 16
            macc = jnp.zeros((16,), jnp.float32)
            for corner in range(8):
                macc = macc + (mw[pl.ds(bo8 + corner * _C + j16, 16)]
                               * mrows[pl.ds(bo8 + corner * _C + j16, 16)])
            mbuf[pl.ds(j16, 16)] = 1.0 / (1.0 + jnp.exp(-macc))
            return c2

        lax.fori_loop(0, _NJ, m_body, 0)

        ocol = (g & 3) * _C

        def acc_level(l, c2):
            def acc_j(j, c3):
                j16 = j * 16
                mv = mbuf[pl.ds(j16, 16)]
                omv = 1.0 - mv
                wt0 = wt[pl.ds(boW + (l * 2) * _C + j16, 16)]
                wt1 = wt[pl.ds(boW + (l * 2 + 1) * _C + j16, 16)]
                a0 = jnp.zeros((16,), jnp.float32)
                a1 = jnp.zeros((16,), jnp.float32)
                b0 = jnp.zeros((16,), jnp.float32)
                b1 = jnp.zeros((16,), jnp.float32)
                for corner in range(8):
                    sb = boS + (l * 8 + corner) * _C + j16
                    wv = w3[pl.ds(sb, 16)]
                    sp = svp[pl.ds(sb, 16)]
                    a0 = a0 + wv * lax.bitcast_convert_type(
                        sp << 16, jnp.float32)
                    a1 = a1 + wv * lax.bitcast_convert_type(
                        sp & _HI, jnp.float32)
                    tb = boT + (l * 16 + corner) * _C + j16
                    tp0 = tvp[pl.ds(tb, 16)]
                    tp1 = tvp[pl.ds(tb + 8 * _C, 16)]
                    r00 = lax.bitcast_convert_type(tp0 << 16, jnp.float32)
                    r01 = lax.bitcast_convert_type(tp0 & _HI, jnp.float32)
                    r10 = lax.bitcast_convert_type(tp1 << 16, jnp.float32)
                    r11 = lax.bitcast_convert_type(tp1 & _HI, jnp.float32)
                    b0 = b0 + wv * (wt0 * r00 + wt1 * r10)
                    b1 = b1 + wv * (wt0 * r01 + wt1 * r11)
                o0 = omv * a0 + mv * b0
                o1 = omv * a1 + mv * b1
                obuf[2 * l, pl.ds(ocol + j16, 16)] = o0
                obuf[2 * l + 1, pl.ds(ocol + j16, 16)] = o1
                return c3

            return lax.fori_loop(0, _NJ, acc_j, c2)

        lax.fori_loop(0, _NL, acc_level, 0)

        @pl.when((g & 3) == 3)
        def _flush():
            base = wid * _PPW + g * _C
            ob = pl.multiple_of(base - 3 * _C, 4 * _C)
            pltpu.sync_copy(obuf, out.at[:, pl.ds(ob, 4 * _C)])

    def chunk(g, carry):
        @pl.when(g < _NCHUNK)
        def _prep():
            idx_phase(g)

        @pl.when(g > 0)
        def _drain():
            drain(g - 1)

        @pl.when(g < _NCHUNK)
        def _fire():
            fire(g)

        @pl.when(g > 0)
        def _acc():
            compute_phase(g - 1)

        return carry

    lax.fori_loop(0, _NCHUNK + 1, chunk, 0)


def _pack_pairs(table):
    c0 = jax.lax.bitcast_convert_type(
        table[:, 0].astype(jnp.bfloat16), jnp.uint16).astype(jnp.uint32)
    c1 = jax.lax.bitcast_convert_type(
        table[:, 1].astype(jnp.bfloat16), jnp.uint16).astype(jnp.uint32)
    return c0 | (c1 << 16)


def kernel(inputs, sembeddings, tembeddings, membeddings):
    x0 = inputs[:, 0]
    x1 = inputs[:, 1]
    x2 = inputs[:, 2]
    x3 = inputs[:, 3]
    spk = _pack_pairs(sembeddings)
    tpk = _pack_pairs(tembeddings)
    cm = _encode(x0, x1, x2, x3, spk, tpk, membeddings)
    return cm.T
